# baseline (speedup 1.0000x reference)
import jax
import jax.numpy as jnp
from jax import lax
from jax.experimental import pallas as pl
from jax.experimental.pallas import tpu as pltpu


def _body(idx_ref, we_ref, wut_ref, out_ref, emb_ref, embt_ref, sem):
    S = emb_ref.shape[0]

    @pl.when(pl.program_id(0) == 0)
    def _gather_issue():
        def issue(i, _):
            row = idx_ref[0, i]
            pltpu.make_async_copy(
                we_ref.at[pl.ds(row, 1)], emb_ref.at[pl.ds(i, 1)], sem
            ).start()
            return _

        lax.fori_loop(0, S, issue, 0, unroll=8)

    @pl.when(pl.program_id(0) == 1)
    def _drain_and_transpose():
        pltpu.make_async_copy(we_ref.at[pl.ds(0, S)], emb_ref, sem).wait()
        embt_ref[...] = emb_ref[...].T.astype(jnp.bfloat16)

    out_ref[...] = jnp.dot(
        wut_ref[...].astype(jnp.bfloat16),
        embt_ref[...],
        preferred_element_type=jnp.float32,
    )


def kernel(x, W_E, W_U):
    B, S = x.shape
    V, D = W_E.shape
    M = B * S
    v_blk = 2000
    out_t = pl.pallas_call(
        _body,
        grid=(V // v_blk + 1,),
        in_specs=[
            pl.BlockSpec(memory_space=pltpu.SMEM),
            pl.BlockSpec(memory_space=pl.ANY),
            pl.BlockSpec((v_blk, D), lambda n: (jnp.maximum(n - 1, 0), 0)),
        ],
        out_specs=pl.BlockSpec(
            (v_blk, M), lambda n: (jnp.maximum(n - 1, 0), 0)
        ),
        out_shape=jax.ShapeDtypeStruct((V, M), jnp.float32),
        scratch_shapes=[
            pltpu.VMEM((M, D), jnp.float32),
            pltpu.VMEM((D, M), jnp.bfloat16),
            pltpu.SemaphoreType.DMA,
        ],
        compiler_params=pltpu.CompilerParams(
            dimension_semantics=("arbitrary",),
        ),
    )(x.astype(jnp.int32), W_E, W_U.T)
    return out_t.T.reshape(B, S, V)


# speedup vs baseline: 1.0167x; 1.0167x over previous
import jax
import jax.numpy as jnp
from jax import lax
from jax.experimental import pallas as pl
from jax.experimental.pallas import tpu as pltpu


def _body(idx_ref, we_ref, wut_ref, out_ref, emb_ref, embt_ref, sem):
    S = emb_ref.shape[0]

    @pl.when(pl.program_id(0) == 0)
    def _gather_and_transpose():
        def issue(i, _):
            row = idx_ref[0, i]
            pltpu.make_async_copy(
                we_ref.at[pl.ds(row, 1)], emb_ref.at[pl.ds(i, 1)], sem
            ).start()
            return _

        lax.fori_loop(0, S, issue, 0, unroll=8)
        pltpu.make_async_copy(we_ref.at[pl.ds(0, S)], emb_ref, sem).wait()
        embt_ref[...] = emb_ref[...].T

    out_ref[...] = lax.dot_general(
        wut_ref[...],
        embt_ref[...],
        (((1,), (0,)), ((), ())),
        precision=lax.Precision.DEFAULT,
        preferred_element_type=jnp.float32,
    )


def kernel(x, W_E, W_U):
    B, S = x.shape
    V, D = W_E.shape
    M = B * S
    v_blk = 2000
    out_t = pl.pallas_call(
        _body,
        grid=(V // v_blk,),
        in_specs=[
            pl.BlockSpec(memory_space=pltpu.SMEM),
            pl.BlockSpec(memory_space=pl.ANY),
            pl.BlockSpec((v_blk, D), lambda n: (n, 0)),
        ],
        out_specs=pl.BlockSpec((v_blk, M), lambda n: (n, 0)),
        out_shape=jax.ShapeDtypeStruct((V, M), jnp.float32),
        scratch_shapes=[
            pltpu.VMEM((M, D), jnp.float32),
            pltpu.VMEM((D, M), jnp.float32),
            pltpu.SemaphoreType.DMA,
        ],
        compiler_params=pltpu.CompilerParams(
            dimension_semantics=("arbitrary",),
        ),
    )(x.astype(jnp.int32), W_E, W_U.T)
    return out_t.T.reshape(B, S, V)


# speedup vs baseline: 1.0197x; 1.0030x over previous
import jax
import jax.numpy as jnp
from jax import lax
from jax.experimental import pallas as pl
from jax.experimental.pallas import tpu as pltpu


def _body(idx_ref, we_ref, wut_ref, out_ref, emb_ref, sem):
    S = emb_ref.shape[0]

    @pl.when(pl.program_id(0) == 0)
    def _gather_and_transpose():
        def issue(i, _):
            row = idx_ref[0, i]
            pltpu.make_async_copy(
                we_ref.at[pl.ds(row, 1)], emb_ref.at[pl.ds(i, 1)], sem
            ).start()
            return _

        lax.fori_loop(0, S, issue, 0, unroll=8)
        pltpu.make_async_copy(we_ref.at[pl.ds(0, S)], emb_ref, sem).wait()

    out_ref[...] = lax.dot_general(
        wut_ref[...],
        emb_ref[...],
        (((1,), (1,)), ((), ())),
        precision=lax.Precision.DEFAULT,
        preferred_element_type=jnp.float32,
    )


def kernel(x, W_E, W_U):
    B, S = x.shape
    V, D = W_E.shape
    M = B * S
    v_blk = 2000
    out_t = pl.pallas_call(
        _body,
        grid=(V // v_blk,),
        in_specs=[
            pl.BlockSpec(memory_space=pltpu.SMEM),
            pl.BlockSpec(memory_space=pl.ANY),
            pl.BlockSpec((v_blk, D), lambda n: (n, 0)),
        ],
        out_specs=pl.BlockSpec((v_blk, M), lambda n: (n, 0)),
        out_shape=jax.ShapeDtypeStruct((V, M), jnp.float32),
        scratch_shapes=[
            pltpu.VMEM((M, D), jnp.float32),
            pltpu.SemaphoreType.DMA,
        ],
        compiler_params=pltpu.CompilerParams(
            dimension_semantics=("arbitrary",),
        ),
    )(x.astype(jnp.int32), W_E, W_U.T)
    return out_t.T.reshape(B, S, V)


# speedup vs baseline: 1.0211x; 1.0013x over previous
import jax
import jax.numpy as jnp
from jax import lax
from jax.experimental import pallas as pl
from jax.experimental.pallas import tpu as pltpu


def _body(idx_ref, we_ref, wut_ref, out_ref, emb_ref, sem):
    S = emb_ref.shape[0]

    @pl.when(pl.program_id(0) == 0)
    def _gather_and_transpose():
        def issue(i, _):
            row = idx_ref[0, i]
            pltpu.make_async_copy(
                we_ref.at[pl.ds(row, 1)], emb_ref.at[pl.ds(i, 1)], sem
            ).start()
            return _

        lax.fori_loop(0, S, issue, 0, unroll=16)
        pltpu.make_async_copy(we_ref.at[pl.ds(0, S)], emb_ref, sem).wait()

    out_ref[...] = lax.dot_general(
        wut_ref[...],
        emb_ref[...],
        (((1,), (1,)), ((), ())),
        precision=lax.Precision.DEFAULT,
        preferred_element_type=jnp.float32,
    )


def kernel(x, W_E, W_U):
    B, S = x.shape
    V, D = W_E.shape
    M = B * S
    v_blk = 2000
    out_t = pl.pallas_call(
        _body,
        grid=(V // v_blk,),
        in_specs=[
            pl.BlockSpec(memory_space=pltpu.SMEM),
            pl.BlockSpec(memory_space=pl.ANY),
            pl.BlockSpec((v_blk, D), lambda n: (n, 0)),
        ],
        out_specs=pl.BlockSpec((v_blk, M), lambda n: (n, 0)),
        out_shape=jax.ShapeDtypeStruct((V, M), jnp.float32),
        scratch_shapes=[
            pltpu.VMEM((M, D), jnp.float32),
            pltpu.SemaphoreType.DMA,
        ],
        compiler_params=pltpu.CompilerParams(
            dimension_semantics=("arbitrary",),
        ),
    )(x.astype(jnp.int32), W_E, W_U.T)
    return out_t.T.reshape(B, S, V)
